# Initial kernel scaffold; baseline (speedup 1.0000x reference)
import jax
import jax.numpy as jnp
from jax.experimental import pallas as pl


def kernel(x, edge_index, pos, in_w1, in_b1, in_w2, in_b2, pos_w1, pos_b1, pos_w2, pos_b2, attn, g_gamma, g_beta, t_gamma, t_beta, tconv_w, tconv_b, out_w1, out_b1, out_w2, out_b2):
    raise NotImplementedError("write your pallas kernel here")


# speedup vs baseline: 22.1047x; 22.1047x over previous
import functools

import jax
import jax.numpy as jnp
from jax import lax
from jax.experimental import pallas as pl
from jax.experimental.pallas import tpu as pltpu
from jax.experimental.pallas import tpu_sc as plsc

N = 10000
N_PAD = 10240
E = 160000
P = 12
D = 64
EPS = 1e-5

NC = 2
NS = 16
ROWS_PER_TILE = N_PAD // NS
P_PER_CORE = P // NC

CONV_EDGES_PER_TILE = E // NS
CONV_BATCH = 400
CONV_NBATCH = CONV_EDGES_PER_TILE // CONV_BATCH
ZCHUNK = 128
FCHUNK = 320

DEG_W = 16
DEG_EDGES_PER_TILE = E // (NC * NS)
DEG_BATCH = 1000
DEG_NBATCH = DEG_EDGES_PER_TILE // DEG_BATCH

@functools.lru_cache(maxsize=None)
def _sc_mesh():
    return plsc.VectorSubcoreMesh(core_axis_name="c", subcore_axis_name="s",
                                  num_cores=NC, num_subcores=NS)


def _mm_t(a, w):
    return lax.dot_general(a, w, (((1,), (1,)), ((), ())),
                           preferred_element_type=jnp.float32)


def _sc_deg_body(col_hbm, out_hbm, ones_v, idx_v, buf_v, acc_sh, sem):
    c = lax.axis_index("c")
    s = lax.axis_index("s")

    def _fill(i, carry):
        ones_v[i, :] = jnp.ones((DEG_W,), jnp.float32)
        buf_v[i % ROWS_PER_TILE, :] = jnp.zeros((DEG_W,), jnp.float32)
        return carry
    lax.fori_loop(0, DEG_BATCH, _fill, 0)

    pltpu.sync_copy(buf_v, acc_sh.at[pl.ds(s * ROWS_PER_TILE, ROWS_PER_TILE)])
    plsc.subcore_barrier()

    base = (c * NS + s) * DEG_EDGES_PER_TILE

    def _batch(b, carry):
        pltpu.sync_copy(col_hbm.at[pl.ds(base + b * DEG_BATCH, DEG_BATCH)],
                        idx_v)
        pltpu.sync_copy(ones_v, acc_sh.at[idx_v], add=True)
        return carry
    lax.fori_loop(0, DEG_NBATCH, _batch, 0)

    plsc.subcore_barrier()
    pltpu.sync_copy(acc_sh.at[pl.ds(s * ROWS_PER_TILE, ROWS_PER_TILE)], buf_v)
    pltpu.sync_copy(
        buf_v, out_hbm.at[pl.ds(c * N_PAD + s * ROWS_PER_TILE, ROWS_PER_TILE)])


@functools.lru_cache(maxsize=None)
def _sc_deg_call():
    return pl.kernel(
        _sc_deg_body,
        out_type=jax.ShapeDtypeStruct((NC * N_PAD, DEG_W), jnp.float32),
        mesh=_sc_mesh(),
        scratch_types=[
            pltpu.VMEM((DEG_BATCH, DEG_W), jnp.float32),
            pltpu.VMEM((DEG_BATCH,), jnp.int32),
            pltpu.VMEM((ROWS_PER_TILE, DEG_W), jnp.float32),
            pltpu.VMEM_SHARED((N_PAD, DEG_W), jnp.float32),
            pltpu.SemaphoreType.DMA,
        ],
        compiler_params=pltpu.CompilerParams(use_tc_tiling_on_sc=False),
    )


def _sc_deg(col):
    return _sc_deg_call()(col)


def _sc_conv_body(table_hbm, row_hbm, col_hbm, out_hbm,
                  ridx_v, cidx_v, rows_v, zero_v, buf_v, acc_sh, sem):
    c = lax.axis_index("c")
    s = lax.axis_index("s")

    def _fill(i, carry):
        for j in range(D // 16):
            zero_v[i, pl.ds(j * 16, 16)] = jnp.zeros((16,), jnp.float32)
        return carry
    lax.fori_loop(0, ZCHUNK, _fill, 0)

    ebase = s * CONV_EDGES_PER_TILE

    def _per_p(pj, carry):
        p = c * P_PER_CORE + pj
        poff = p * N

        def _zero(z, zc):
            pltpu.sync_copy(
                zero_v,
                acc_sh.at[pl.ds(s * ROWS_PER_TILE + z * ZCHUNK, ZCHUNK)])
            return zc
        lax.fori_loop(0, ROWS_PER_TILE // ZCHUNK, _zero, 0)
        plsc.subcore_barrier()

        def _batch(b, bc):
            off = ebase + b * CONV_BATCH
            pltpu.sync_copy(row_hbm.at[pl.ds(off, CONV_BATCH)], ridx_v)
            pltpu.sync_copy(col_hbm.at[pl.ds(off, CONV_BATCH)], cidx_v)

            def _shift(i, ic):
                ridx_v[pl.ds(i * 16, 16)] = ridx_v[pl.ds(i * 16, 16)] + poff
                return ic
            lax.fori_loop(0, CONV_BATCH // 16, _shift, 0)

            pltpu.async_copy(table_hbm.at[ridx_v], rows_v, sem).wait()
            pltpu.sync_copy(rows_v, acc_sh.at[cidx_v], add=True)
            return bc
        lax.fori_loop(0, CONV_NBATCH, _batch, 0)

        plsc.subcore_barrier()

        def _flush(f, fc):
            r0 = s * ROWS_PER_TILE + f * FCHUNK
            pltpu.sync_copy(acc_sh.at[pl.ds(r0, FCHUNK)], buf_v)
            pltpu.sync_copy(buf_v, out_hbm.at[pl.ds(p * N_PAD + r0, FCHUNK)])
            return fc
        lax.fori_loop(0, ROWS_PER_TILE // FCHUNK, _flush, 0)
        return carry
    lax.fori_loop(0, P_PER_CORE, _per_p, 0)


@functools.lru_cache(maxsize=None)
def _sc_conv_call():
    return pl.kernel(
        _sc_conv_body,
        out_type=jax.ShapeDtypeStruct((P * N_PAD, D), jnp.float32),
        mesh=_sc_mesh(),
        scratch_types=[
            pltpu.VMEM((CONV_BATCH,), jnp.int32),
            pltpu.VMEM((CONV_BATCH,), jnp.int32),
            pltpu.VMEM((CONV_BATCH, D), jnp.float32),
            pltpu.VMEM((ZCHUNK, D), jnp.float32),
            pltpu.VMEM((FCHUNK, D), jnp.float32),
            pltpu.VMEM_SHARED((N_PAD, D), jnp.float32),
            pltpu.SemaphoreType.DMA,
        ],
        compiler_params=pltpu.CompilerParams(use_tc_tiling_on_sc=False),
    )


def _sc_conv(table, row, col):
    return _sc_conv_call()(table, row, col)


def _k1_body(x_ref, pos_ref, iw1_ref, ib1_ref, iw2_ref, ib2_ref,
             pw1_ref, pb1_ref, pw2_ref, pb2_ref, out_ref, posc):
    pid = pl.program_id(0)

    @pl.when(pid == 0)
    def _():
        ph = jax.nn.relu(_mm_t(pos_ref[...], pw1_ref[...]) + pb1_ref[...])
        posc[...] = _mm_t(ph, pw2_ref[...]) + pb2_ref[...]

    xb = x_ref[0]
    h = jax.nn.relu(_mm_t(xb, iw1_ref[...]) + ib1_ref[...])
    h = _mm_t(h, iw2_ref[...]) + ib2_ref[...]
    out_ref[0] = h + posc[...]


def _k1(x, pos, iw1, ib1, iw2, ib2, pw1, pb1, pw2, pb2):
    full = lambda *shape: pl.BlockSpec(shape, lambda p: (0,) * len(shape))
    return pl.pallas_call(
        _k1_body,
        grid=(P,),
        in_specs=[
            pl.BlockSpec((1, N, D), lambda p: (p, 0, 0)),
            full(N, D), full(D, D), full(D,), full(D, D), full(D,),
            full(D, D), full(D,), full(D, D), full(D,),
        ],
        out_specs=pl.BlockSpec((1, N, D), lambda p: (p, 0, 0)),
        out_shape=jax.ShapeDtypeStruct((P, N, D), jnp.float32),
        scratch_shapes=[pltpu.VMEM((N, D), jnp.float32)],
    )(x, pos, iw1, ib1, iw2, ib2, pw1, pb1, pw2, pb2)


def _dinv_from(degp):
    deg = degp[0, :N, 0] + degp[1, :N, 0]
    return jnp.where(deg > 0, lax.rsqrt(jnp.maximum(deg, 1.0)), 0.0)


def _ln_scale_body(h_ref, g_ref, b_ref, degp_ref, out_ref):
    hb = h_ref[0]
    mean = jnp.mean(hb)
    var = jnp.mean((hb - mean) ** 2)
    y = (hb - mean) * lax.rsqrt(var + EPS) * g_ref[...] + b_ref[...]
    dinv = _dinv_from(degp_ref[...])
    out_ref[0] = y * dinv[:, None]


def _k2(h, gamma, beta, degp):
    full = lambda *shape: pl.BlockSpec(shape, lambda p: (0,) * len(shape))
    return pl.pallas_call(
        _ln_scale_body,
        grid=(P,),
        in_specs=[
            pl.BlockSpec((1, N, D), lambda p: (p, 0, 0)),
            full(N, D), full(N, D), full(NC, N_PAD, DEG_W),
        ],
        out_specs=pl.BlockSpec((1, N, D), lambda p: (p, 0, 0)),
        out_shape=jax.ShapeDtypeStruct((P, N, D), jnp.float32),
    )(h, gamma, beta, degp)


def _scale_ln_body(c_ref, g_ref, b_ref, degp_ref, out_ref):
    dinv = _dinv_from(degp_ref[...])
    sb = c_ref[0, :N, :] * dinv[:, None]
    mean = jnp.mean(sb)
    var = jnp.mean((sb - mean) ** 2)
    out_ref[0] = (sb - mean) * lax.rsqrt(var + EPS) * g_ref[...] + b_ref[...]


def _k4a(cv, gamma, beta, degp):
    full = lambda *shape: pl.BlockSpec(shape, lambda p: (0,) * len(shape))
    return pl.pallas_call(
        _scale_ln_body,
        grid=(P,),
        in_specs=[
            pl.BlockSpec((1, N_PAD, D), lambda p: (p, 0, 0)),
            full(N, D), full(N, D), full(NC, N_PAD, DEG_W),
        ],
        out_specs=pl.BlockSpec((1, N, D), lambda p: (p, 0, 0)),
        out_shape=jax.ShapeDtypeStruct((P, N, D), jnp.float32),
    )(cv, gamma, beta, degp)


_TCONV_NB = 2000


def _tconv_body(t_ref, w_ref, b_ref, out_ref):
    tb = t_ref[...].reshape(P, _TCONV_NB * D)
    o = lax.dot_general(w_ref[...], tb, (((1,), (0,)), ((), ())),
                        preferred_element_type=jnp.float32)
    out_ref[...] = o.reshape(P, _TCONV_NB, D) + b_ref[...][:, None, None]


def _k4b(t, w, b):
    return pl.pallas_call(
        _tconv_body,
        grid=(N // _TCONV_NB,),
        in_specs=[
            pl.BlockSpec((P, _TCONV_NB, D), lambda n: (0, n, 0)),
            pl.BlockSpec((P, P), lambda n: (0, 0)),
            pl.BlockSpec((P,), lambda n: (0,)),
        ],
        out_specs=pl.BlockSpec((P, _TCONV_NB, D), lambda n: (0, n, 0)),
        out_shape=jax.ShapeDtypeStruct((P, N, D), jnp.float32),
    )(t, w, b)


def _k5_body(h0_ref, h1_ref, h2_ref, attn_ref, w1_ref, b1_ref, w2_ref, b2_ref,
             out_ref):
    a = jax.nn.softmax(attn_ref[...], axis=0)
    h = h0_ref[0] * a[0, 0] + h1_ref[0] * a[1, 0] + h2_ref[0] * a[2, 0]
    h1 = jax.nn.relu(_mm_t(h, w1_ref[...]) + b1_ref[...])
    out_ref[0] = _mm_t(h1, w2_ref[...]) + b2_ref[...]


def _k5(h0, h1, h2, attn, w1, b1, w2, b2):
    full = lambda *shape: pl.BlockSpec(shape, lambda p: (0,) * len(shape))
    blk = pl.BlockSpec((1, N, D), lambda p: (p, 0, 0))
    return pl.pallas_call(
        _k5_body,
        grid=(P,),
        in_specs=[blk, blk, blk, full(3, 1),
                  full(D, D), full(D,), full(D, D), full(D,)],
        out_specs=blk,
        out_shape=jax.ShapeDtypeStruct((P, N, D), jnp.float32),
    )(h0, h1, h2, attn, w1, b1, w2, b2)


def kernel(x, edge_index, pos, in_w1, in_b1, in_w2, in_b2, pos_w1, pos_b1,
           pos_w2, pos_b2, attn, g_gamma, g_beta, t_gamma, t_beta, tconv_w,
           tconv_b, out_w1, out_b1, out_w2, out_b2):
    row = edge_index[0]
    col = edge_index[1]

    degp = _sc_deg(col).reshape(NC, N_PAD, DEG_W)

    h = _k1(x.reshape(P, N, D), pos,
            in_w1, in_b1, in_w2, in_b2, pos_w1, pos_b1, pos_w2, pos_b2)
    skips = [h]
    for i in range(g_gamma.shape[0]):
        g = _k2(h, g_gamma[i], g_beta[i], degp)
        cv = _sc_conv(g.reshape(P * N, D), row, col)
        t = _k4a(cv.reshape(P, N_PAD, D), t_gamma[i], t_beta[i], degp)
        h = _k4b(t, tconv_w[i], tconv_b[i])
        skips.append(h)

    return _k5(skips[0], skips[1], skips[2], attn,
               out_w1, out_b1, out_w2, out_b2)


# speedup vs baseline: 28.0871x; 1.2706x over previous
import functools

import jax
import jax.numpy as jnp
from jax import lax
from jax.experimental import pallas as pl
from jax.experimental.pallas import tpu as pltpu
from jax.experimental.pallas import tpu_sc as plsc

N = 10000
N_PAD = 10240
E = 160000
P = 12
D = 64
EPS = 1e-5

NC = 2
NS = 16
ROWS_PER_TILE = N_PAD // NS
P_PER_CORE = P // NC

CONV_EDGES_PER_TILE = E // NS
CONV_BATCH = 400
CONV_NBATCH = CONV_EDGES_PER_TILE // CONV_BATCH
ZCHUNK = 128
FCHUNK = 320

DEG_W = 16
DEG_EDGES_PER_TILE = E // (NC * NS)
DEG_BATCH = 1000
DEG_NBATCH = DEG_EDGES_PER_TILE // DEG_BATCH

@functools.lru_cache(maxsize=None)
def _sc_mesh():
    return plsc.VectorSubcoreMesh(core_axis_name="c", subcore_axis_name="s",
                                  num_cores=NC, num_subcores=NS)


def _mm_t(a, w):
    return lax.dot_general(a, w, (((1,), (1,)), ((), ())),
                           preferred_element_type=jnp.float32)


def _sc_deg_body(col_hbm, out_hbm, ones_v, idx_v, buf_v, acc_sh, sem):
    c = lax.axis_index("c")
    s = lax.axis_index("s")

    def _fill(i, carry):
        ones_v[i, :] = jnp.ones((DEG_W,), jnp.float32)
        buf_v[i % ROWS_PER_TILE, :] = jnp.zeros((DEG_W,), jnp.float32)
        return carry
    lax.fori_loop(0, DEG_BATCH, _fill, 0)

    pltpu.sync_copy(buf_v, acc_sh.at[pl.ds(s * ROWS_PER_TILE, ROWS_PER_TILE)])
    plsc.subcore_barrier()

    base = (c * NS + s) * DEG_EDGES_PER_TILE

    def _batch(b, carry):
        pltpu.sync_copy(col_hbm.at[pl.ds(base + b * DEG_BATCH, DEG_BATCH)],
                        idx_v)
        pltpu.sync_copy(ones_v, acc_sh.at[idx_v], add=True)
        return carry
    lax.fori_loop(0, DEG_NBATCH, _batch, 0)

    plsc.subcore_barrier()
    pltpu.sync_copy(acc_sh.at[pl.ds(s * ROWS_PER_TILE, ROWS_PER_TILE)], buf_v)
    pltpu.sync_copy(
        buf_v, out_hbm.at[pl.ds(c * N_PAD + s * ROWS_PER_TILE, ROWS_PER_TILE)])


@functools.lru_cache(maxsize=None)
def _sc_deg_call():
    return pl.kernel(
        _sc_deg_body,
        out_type=jax.ShapeDtypeStruct((NC * N_PAD, DEG_W), jnp.float32),
        mesh=_sc_mesh(),
        scratch_types=[
            pltpu.VMEM((DEG_BATCH, DEG_W), jnp.float32),
            pltpu.VMEM((DEG_BATCH,), jnp.int32),
            pltpu.VMEM((ROWS_PER_TILE, DEG_W), jnp.float32),
            pltpu.VMEM_SHARED((N_PAD, DEG_W), jnp.float32),
            pltpu.SemaphoreType.DMA,
        ],
        compiler_params=pltpu.CompilerParams(use_tc_tiling_on_sc=False),
    )


def _sc_deg(col):
    return _sc_deg_call()(col)


def _sc_conv_body(table_hbm, row_hbm, col_hbm, out_hbm,
                  ridx_v, cidx_v, rows_v, zero_v, buf_v, acc_sh, sem0, sem1):
    c = lax.axis_index("c")
    s = lax.axis_index("s")
    sems = (sem0, sem1)

    def _fill(i, carry):
        for j in range(D // 16):
            zero_v[i, pl.ds(j * 16, 16)] = jnp.zeros((16,), jnp.float32)
        return carry
    lax.fori_loop(0, ZCHUNK, _fill, 0)

    ebase = s * CONV_EDGES_PER_TILE

    def _per_p(pj, carry):
        p = c * P_PER_CORE + pj
        poff = p * N

        def _zero(z, zc):
            pltpu.sync_copy(
                zero_v,
                acc_sh.at[pl.ds(s * ROWS_PER_TILE + z * ZCHUNK, ZCHUNK)])
            return zc
        lax.fori_loop(0, ROWS_PER_TILE // ZCHUNK, _zero, 0)
        plsc.subcore_barrier()

        def _fire(b, par):
            off = ebase + b * CONV_BATCH
            pltpu.sync_copy(row_hbm.at[pl.ds(off, CONV_BATCH)],
                            ridx_v.at[par])
            pltpu.sync_copy(col_hbm.at[pl.ds(off, CONV_BATCH)],
                            cidx_v.at[par])

            def _shift(i, ic):
                ridx_v[par, pl.ds(i * 16, 16)] = (
                    ridx_v[par, pl.ds(i * 16, 16)] + poff)
                return ic
            lax.fori_loop(0, CONV_BATCH // 16, _shift, 0)
            pltpu.async_copy(table_hbm.at[ridx_v.at[par]], rows_v.at[par],
                             sems[par])

        def _drain(par):
            pltpu.make_async_copy(table_hbm.at[ridx_v.at[par]],
                                  rows_v.at[par], sems[par]).wait()
            pltpu.sync_copy(rows_v.at[par], acc_sh.at[cidx_v.at[par]],
                            add=True)

        _fire(0, 0)

        def _pair(k, bc):
            _fire(2 * k + 1, 1)
            _drain(0)
            _fire(2 * k + 2, 0)
            _drain(1)
            return bc
        lax.fori_loop(0, (CONV_NBATCH - 1) // 2, _pair, 0)
        _drain(0)

        plsc.subcore_barrier()

        def _flush(f, fc):
            r0 = s * ROWS_PER_TILE + f * FCHUNK
            pltpu.sync_copy(acc_sh.at[pl.ds(r0, FCHUNK)], buf_v)
            pltpu.sync_copy(buf_v, out_hbm.at[pl.ds(p * N_PAD + r0, FCHUNK)])
            return fc
        lax.fori_loop(0, ROWS_PER_TILE // FCHUNK, _flush, 0)
        return carry
    lax.fori_loop(0, P_PER_CORE, _per_p, 0)


@functools.lru_cache(maxsize=None)
def _sc_conv_call():
    return pl.kernel(
        _sc_conv_body,
        out_type=jax.ShapeDtypeStruct((P * N_PAD, D), jnp.float32),
        mesh=_sc_mesh(),
        scratch_types=[
            pltpu.VMEM((2, CONV_BATCH), jnp.int32),
            pltpu.VMEM((2, CONV_BATCH), jnp.int32),
            pltpu.VMEM((2, CONV_BATCH, D), jnp.float32),
            pltpu.VMEM((ZCHUNK, D), jnp.float32),
            pltpu.VMEM((FCHUNK, D), jnp.float32),
            pltpu.VMEM_SHARED((N_PAD, D), jnp.float32),
            pltpu.SemaphoreType.DMA,
            pltpu.SemaphoreType.DMA,
        ],
        compiler_params=pltpu.CompilerParams(use_tc_tiling_on_sc=False),
    )


def _sc_conv(table, row, col):
    return _sc_conv_call()(table, row, col)


def _k1_body(x_ref, pos_ref, iw1_ref, ib1_ref, iw2_ref, ib2_ref,
             pw1_ref, pb1_ref, pw2_ref, pb2_ref, out_ref, posc):
    pid = pl.program_id(0)

    @pl.when(pid == 0)
    def _():
        ph = jax.nn.relu(_mm_t(pos_ref[...], pw1_ref[...]) + pb1_ref[...])
        posc[...] = _mm_t(ph, pw2_ref[...]) + pb2_ref[...]

    xb = x_ref[0]
    h = jax.nn.relu(_mm_t(xb, iw1_ref[...]) + ib1_ref[...])
    h = _mm_t(h, iw2_ref[...]) + ib2_ref[...]
    out_ref[0] = h + posc[...]


def _k1(x, pos, iw1, ib1, iw2, ib2, pw1, pb1, pw2, pb2):
    full = lambda *shape: pl.BlockSpec(shape, lambda p: (0,) * len(shape))
    return pl.pallas_call(
        _k1_body,
        grid=(P,),
        in_specs=[
            pl.BlockSpec((1, N, D), lambda p: (p, 0, 0)),
            full(N, D), full(D, D), full(D,), full(D, D), full(D,),
            full(D, D), full(D,), full(D, D), full(D,),
        ],
        out_specs=pl.BlockSpec((1, N, D), lambda p: (p, 0, 0)),
        out_shape=jax.ShapeDtypeStruct((P, N, D), jnp.float32),
        scratch_shapes=[pltpu.VMEM((N, D), jnp.float32)],
    )(x, pos, iw1, ib1, iw2, ib2, pw1, pb1, pw2, pb2)


def _dinv_from(degp):
    deg = degp[0, :N, 0] + degp[1, :N, 0]
    return jnp.where(deg > 0, lax.rsqrt(jnp.maximum(deg, 1.0)), 0.0)


def _ln_scale_body(h_ref, g_ref, b_ref, degp_ref, out_ref):
    hb = h_ref[0]
    mean = jnp.mean(hb)
    var = jnp.mean((hb - mean) ** 2)
    y = (hb - mean) * lax.rsqrt(var + EPS) * g_ref[...] + b_ref[...]
    dinv = _dinv_from(degp_ref[...])
    out_ref[0] = y * dinv[:, None]


def _k2(h, gamma, beta, degp):
    full = lambda *shape: pl.BlockSpec(shape, lambda p: (0,) * len(shape))
    return pl.pallas_call(
        _ln_scale_body,
        grid=(P,),
        in_specs=[
            pl.BlockSpec((1, N, D), lambda p: (p, 0, 0)),
            full(N, D), full(N, D), full(NC, N_PAD, DEG_W),
        ],
        out_specs=pl.BlockSpec((1, N, D), lambda p: (p, 0, 0)),
        out_shape=jax.ShapeDtypeStruct((P, N, D), jnp.float32),
    )(h, gamma, beta, degp)


def _scale_ln_body(c_ref, g_ref, b_ref, degp_ref, out_ref):
    dinv = _dinv_from(degp_ref[...])
    sb = c_ref[0, :N, :] * dinv[:, None]
    mean = jnp.mean(sb)
    var = jnp.mean((sb - mean) ** 2)
    out_ref[0] = (sb - mean) * lax.rsqrt(var + EPS) * g_ref[...] + b_ref[...]


def _k4a(cv, gamma, beta, degp):
    full = lambda *shape: pl.BlockSpec(shape, lambda p: (0,) * len(shape))
    return pl.pallas_call(
        _scale_ln_body,
        grid=(P,),
        in_specs=[
            pl.BlockSpec((1, N_PAD, D), lambda p: (p, 0, 0)),
            full(N, D), full(N, D), full(NC, N_PAD, DEG_W),
        ],
        out_specs=pl.BlockSpec((1, N, D), lambda p: (p, 0, 0)),
        out_shape=jax.ShapeDtypeStruct((P, N, D), jnp.float32),
    )(cv, gamma, beta, degp)


_TCONV_NB = 2000


def _tconv_body(t_ref, w_ref, b_ref, out_ref):
    tb = t_ref[...].reshape(P, _TCONV_NB * D)
    o = lax.dot_general(w_ref[...], tb, (((1,), (0,)), ((), ())),
                        preferred_element_type=jnp.float32)
    out_ref[...] = o.reshape(P, _TCONV_NB, D) + b_ref[...][:, None, None]


def _k4b(t, w, b):
    return pl.pallas_call(
        _tconv_body,
        grid=(N // _TCONV_NB,),
        in_specs=[
            pl.BlockSpec((P, _TCONV_NB, D), lambda n: (0, n, 0)),
            pl.BlockSpec((P, P), lambda n: (0, 0)),
            pl.BlockSpec((P,), lambda n: (0,)),
        ],
        out_specs=pl.BlockSpec((P, _TCONV_NB, D), lambda n: (0, n, 0)),
        out_shape=jax.ShapeDtypeStruct((P, N, D), jnp.float32),
    )(t, w, b)


def _k5_body(h0_ref, h1_ref, h2_ref, attn_ref, w1_ref, b1_ref, w2_ref, b2_ref,
             out_ref):
    a = jax.nn.softmax(attn_ref[...], axis=0)
    h = h0_ref[0] * a[0, 0] + h1_ref[0] * a[1, 0] + h2_ref[0] * a[2, 0]
    h1 = jax.nn.relu(_mm_t(h, w1_ref[...]) + b1_ref[...])
    out_ref[0] = _mm_t(h1, w2_ref[...]) + b2_ref[...]


def _k5(h0, h1, h2, attn, w1, b1, w2, b2):
    full = lambda *shape: pl.BlockSpec(shape, lambda p: (0,) * len(shape))
    blk = pl.BlockSpec((1, N, D), lambda p: (p, 0, 0))
    return pl.pallas_call(
        _k5_body,
        grid=(P,),
        in_specs=[blk, blk, blk, full(3, 1),
                  full(D, D), full(D,), full(D, D), full(D,)],
        out_specs=blk,
        out_shape=jax.ShapeDtypeStruct((P, N, D), jnp.float32),
    )(h0, h1, h2, attn, w1, b1, w2, b2)


def kernel(x, edge_index, pos, in_w1, in_b1, in_w2, in_b2, pos_w1, pos_b1,
           pos_w2, pos_b2, attn, g_gamma, g_beta, t_gamma, t_beta, tconv_w,
           tconv_b, out_w1, out_b1, out_w2, out_b2):
    row = edge_index[0]
    col = edge_index[1]

    degp = _sc_deg(col).reshape(NC, N_PAD, DEG_W)

    h = _k1(x.reshape(P, N, D), pos,
            in_w1, in_b1, in_w2, in_b2, pos_w1, pos_b1, pos_w2, pos_b2)
    skips = [h]
    for i in range(g_gamma.shape[0]):
        g = _k2(h, g_gamma[i], g_beta[i], degp)
        cv = _sc_conv(g.reshape(P * N, D), row, col)
        t = _k4a(cv.reshape(P, N_PAD, D), t_gamma[i], t_beta[i], degp)
        h = _k4b(t, tconv_w[i], tconv_b[i])
        skips.append(h)

    return _k5(skips[0], skips[1], skips[2], attn,
               out_w1, out_b1, out_w2, out_b2)


# speedup vs baseline: 32.7080x; 1.1645x over previous
import functools

import jax
import jax.numpy as jnp
from jax import lax
from jax.experimental import pallas as pl
from jax.experimental.pallas import tpu as pltpu
from jax.experimental.pallas import tpu_sc as plsc

N = 10000
N_PAD = 10240
E = 160000
P = 12
D = 64
EPS = 1e-5

NC = 2
NS = 16
ROWS_PER_TILE = N_PAD // NS
P_PER_CORE = P // NC

CONV_EDGES_PER_TILE = E // NS
CONV_BATCH = 400
CONV_NBATCH = CONV_EDGES_PER_TILE // CONV_BATCH
ZCHUNK = 64
FCHUNK = 160

DEG_W = 16
DEG_EDGES_PER_TILE = E // (NC * NS)
DEG_BATCH = 1000
DEG_NBATCH = DEG_EDGES_PER_TILE // DEG_BATCH

@functools.lru_cache(maxsize=None)
def _sc_mesh():
    return plsc.VectorSubcoreMesh(core_axis_name="c", subcore_axis_name="s",
                                  num_cores=NC, num_subcores=NS)


def _mm_t(a, w):
    return lax.dot_general(a, w, (((1,), (1,)), ((), ())),
                           preferred_element_type=jnp.float32)


def _sc_deg_body(col_hbm, out_hbm, ones_v, idx_v, buf_v, acc_sh, sem):
    c = lax.axis_index("c")
    s = lax.axis_index("s")

    def _fill(i, carry):
        ones_v[i, :] = jnp.ones((DEG_W,), jnp.float32)
        buf_v[i % ROWS_PER_TILE, :] = jnp.zeros((DEG_W,), jnp.float32)
        return carry
    lax.fori_loop(0, DEG_BATCH, _fill, 0)

    pltpu.sync_copy(buf_v, acc_sh.at[pl.ds(s * ROWS_PER_TILE, ROWS_PER_TILE)])
    plsc.subcore_barrier()

    base = (c * NS + s) * DEG_EDGES_PER_TILE

    def _batch(b, carry):
        pltpu.sync_copy(col_hbm.at[pl.ds(base + b * DEG_BATCH, DEG_BATCH)],
                        idx_v)
        pltpu.sync_copy(ones_v, acc_sh.at[idx_v], add=True)
        return carry
    lax.fori_loop(0, DEG_NBATCH, _batch, 0)

    plsc.subcore_barrier()
    pltpu.sync_copy(acc_sh.at[pl.ds(s * ROWS_PER_TILE, ROWS_PER_TILE)], buf_v)
    pltpu.sync_copy(
        buf_v, out_hbm.at[pl.ds(c * N_PAD + s * ROWS_PER_TILE, ROWS_PER_TILE)])


@functools.lru_cache(maxsize=None)
def _sc_deg_call():
    return pl.kernel(
        _sc_deg_body,
        out_type=jax.ShapeDtypeStruct((NC * N_PAD, DEG_W), jnp.float32),
        mesh=_sc_mesh(),
        scratch_types=[
            pltpu.VMEM((DEG_BATCH, DEG_W), jnp.float32),
            pltpu.VMEM((DEG_BATCH,), jnp.int32),
            pltpu.VMEM((ROWS_PER_TILE, DEG_W), jnp.float32),
            pltpu.VMEM_SHARED((N_PAD, DEG_W), jnp.float32),
            pltpu.SemaphoreType.DMA,
        ],
        compiler_params=pltpu.CompilerParams(use_tc_tiling_on_sc=False),
    )


def _sc_deg(col):
    return _sc_deg_call()(col)


def _sc_conv_body(table_hbm, row_hbm, col_hbm, out_hbm,
                  rowc_v, colc_v, gidx_v, cidx_v, rows_v, zero_v, buf_v,
                  acc_sh, sem0, sem1):
    c = lax.axis_index("c")
    s = lax.axis_index("s")
    sems = (sem0, sem1)

    def _fill(i, carry):
        for j in range(D // 16):
            zero_v[i, pl.ds(j * 16, 16)] = jnp.zeros((16,), jnp.float32)
        return carry
    lax.fori_loop(0, ZCHUNK, _fill, 0)

    ebase = s * CONV_EDGES_PER_TILE
    pltpu.sync_copy(row_hbm.at[pl.ds(ebase, CONV_EDGES_PER_TILE)], rowc_v)
    pltpu.sync_copy(col_hbm.at[pl.ds(ebase, CONV_EDGES_PER_TILE)], colc_v)

    def _per_p(pj, carry):
        p = c * P_PER_CORE + pj
        poff = p * N

        def _zero(z, zc):
            pltpu.sync_copy(
                zero_v,
                acc_sh.at[pl.ds(s * ROWS_PER_TILE + z * ZCHUNK, ZCHUNK)])
            return zc
        lax.fori_loop(0, ROWS_PER_TILE // ZCHUNK, _zero, 0)
        plsc.subcore_barrier()

        def _fire(b, par):
            def _prep(i, ic):
                sl = pl.ds(b * CONV_BATCH + i * 16, 16)
                dst = pl.ds(i * 16, 16)
                gidx_v[par, dst] = rowc_v[sl] + poff
                cidx_v[par, dst] = colc_v[sl]
                return ic
            lax.fori_loop(0, CONV_BATCH // 16, _prep, 0)
            pltpu.async_copy(table_hbm.at[gidx_v.at[par]], rows_v.at[par],
                             sems[par])

        def _drain(par):
            pltpu.make_async_copy(table_hbm.at[gidx_v.at[par]],
                                  rows_v.at[par], sems[par]).wait()
            pltpu.sync_copy(rows_v.at[par], acc_sh.at[cidx_v.at[par]],
                            add=True)

        _fire(0, 0)

        def _pair(k, bc):
            _fire(2 * k + 1, 1)
            _drain(0)
            _fire(2 * k + 2, 0)
            _drain(1)
            return bc
        lax.fori_loop(0, (CONV_NBATCH - 1) // 2, _pair, 0)
        _drain(0)

        plsc.subcore_barrier()

        def _flush(f, fc):
            r0 = s * ROWS_PER_TILE + f * FCHUNK
            pltpu.sync_copy(acc_sh.at[pl.ds(r0, FCHUNK)], buf_v)
            pltpu.sync_copy(buf_v, out_hbm.at[pl.ds(p * N_PAD + r0, FCHUNK)])
            return fc
        lax.fori_loop(0, ROWS_PER_TILE // FCHUNK, _flush, 0)
        return carry
    lax.fori_loop(0, P_PER_CORE, _per_p, 0)


@functools.lru_cache(maxsize=None)
def _sc_conv_call():
    return pl.kernel(
        _sc_conv_body,
        out_type=jax.ShapeDtypeStruct((P * N_PAD, D), jnp.float32),
        mesh=_sc_mesh(),
        scratch_types=[
            pltpu.VMEM((CONV_EDGES_PER_TILE,), jnp.int32),
            pltpu.VMEM((CONV_EDGES_PER_TILE,), jnp.int32),
            pltpu.VMEM((2, CONV_BATCH), jnp.int32),
            pltpu.VMEM((2, CONV_BATCH), jnp.int32),
            pltpu.VMEM((2, CONV_BATCH, D), jnp.float32),
            pltpu.VMEM((ZCHUNK, D), jnp.float32),
            pltpu.VMEM((FCHUNK, D), jnp.float32),
            pltpu.VMEM_SHARED((N_PAD, D), jnp.float32),
            pltpu.SemaphoreType.DMA,
            pltpu.SemaphoreType.DMA,
        ],
        compiler_params=pltpu.CompilerParams(use_tc_tiling_on_sc=False),
    )


def _sc_conv(table, row, col):
    return _sc_conv_call()(table, row, col)


def _ln_dinv(hb, gamma, beta, dinv):
    mean = jnp.mean(hb)
    var = jnp.mean((hb - mean) ** 2)
    y = (hb - mean) * lax.rsqrt(var + EPS) * gamma + beta
    return y * dinv[:, None]


def _k12_body(x_ref, pos_ref, iw1_ref, ib1_ref, iw2_ref, ib2_ref,
              pw1_ref, pb1_ref, pw2_ref, pb2_ref, g_ref, b_ref, dinv_ref,
              h_ref, gout_ref, posc):
    pid = pl.program_id(0)

    @pl.when(pid == 0)
    def _():
        ph = jax.nn.relu(_mm_t(pos_ref[...], pw1_ref[...]) + pb1_ref[...])
        posc[...] = _mm_t(ph, pw2_ref[...]) + pb2_ref[...]

    xb = x_ref[0]
    h = jax.nn.relu(_mm_t(xb, iw1_ref[...]) + ib1_ref[...])
    h = _mm_t(h, iw2_ref[...]) + ib2_ref[...] + posc[...]
    h_ref[0] = h
    gout_ref[0] = _ln_dinv(h, g_ref[...], b_ref[...], dinv_ref[...])


def _k12(x, pos, iw1, ib1, iw2, ib2, pw1, pb1, pw2, pb2, gamma, beta, dinv):
    full = lambda *shape: pl.BlockSpec(shape, lambda p: (0,) * len(shape))
    blk = pl.BlockSpec((1, N, D), lambda p: (p, 0, 0))
    return pl.pallas_call(
        _k12_body,
        grid=(P,),
        in_specs=[
            blk,
            full(N, D), full(D, D), full(D,), full(D, D), full(D,),
            full(D, D), full(D,), full(D, D), full(D,),
            full(N, D), full(N, D), full(N,),
        ],
        out_specs=[blk, blk],
        out_shape=[jax.ShapeDtypeStruct((P, N, D), jnp.float32),
                   jax.ShapeDtypeStruct((P, N, D), jnp.float32)],
        scratch_shapes=[pltpu.VMEM((N, D), jnp.float32)],
        compiler_params=pltpu.CompilerParams(
            vmem_limit_bytes=100 * 1024 * 1024),
    )(x, pos, iw1, ib1, iw2, ib2, pw1, pb1, pw2, pb2, gamma, beta, dinv)


def _dinv_body(degp_ref, out_ref):
    deg = degp_ref[0, :N, 0] + degp_ref[1, :N, 0]
    out_ref[...] = jnp.where(deg > 0, lax.rsqrt(jnp.maximum(deg, 1.0)), 0.0)


def _kdinv(degp):
    return pl.pallas_call(
        _dinv_body,
        in_specs=[pl.BlockSpec((NC, N_PAD, DEG_W), lambda: (0, 0, 0))],
        out_specs=pl.BlockSpec((N,), lambda: (0,)),
        out_shape=jax.ShapeDtypeStruct((N,), jnp.float32),
    )(degp)


def _ln_scale_body(h_ref, g_ref, b_ref, dinv_ref, out_ref):
    out_ref[0] = _ln_dinv(h_ref[0], g_ref[...], b_ref[...], dinv_ref[...])


def _k2(h, gamma, beta, dinv):
    full = lambda *shape: pl.BlockSpec(shape, lambda p: (0,) * len(shape))
    return pl.pallas_call(
        _ln_scale_body,
        grid=(P,),
        in_specs=[
            pl.BlockSpec((1, N, D), lambda p: (p, 0, 0)),
            full(N, D), full(N, D), full(N,),
        ],
        out_specs=pl.BlockSpec((1, N, D), lambda p: (p, 0, 0)),
        out_shape=jax.ShapeDtypeStruct((P, N, D), jnp.float32),
    )(h, gamma, beta, dinv)


def _scale_ln_body(c_ref, g_ref, b_ref, dinv_ref, out_ref):
    sb = c_ref[0, :N, :] * dinv_ref[...][:, None]
    mean = jnp.mean(sb)
    var = jnp.mean((sb - mean) ** 2)
    out_ref[0] = (sb - mean) * lax.rsqrt(var + EPS) * g_ref[...] + b_ref[...]


def _k4a(cv, gamma, beta, dinv):
    full = lambda *shape: pl.BlockSpec(shape, lambda p: (0,) * len(shape))
    return pl.pallas_call(
        _scale_ln_body,
        grid=(P,),
        in_specs=[
            pl.BlockSpec((1, N_PAD, D), lambda p: (p, 0, 0)),
            full(N, D), full(N, D), full(N,),
        ],
        out_specs=pl.BlockSpec((1, N, D), lambda p: (p, 0, 0)),
        out_shape=jax.ShapeDtypeStruct((P, N, D), jnp.float32),
    )(cv, gamma, beta, dinv)


_TCONV_NB = 2000


def _tconv_body(t_ref, w_ref, b_ref, out_ref):
    tb = t_ref[...].reshape(P, _TCONV_NB * D)
    o = lax.dot_general(w_ref[...], tb, (((1,), (0,)), ((), ())),
                        preferred_element_type=jnp.float32)
    out_ref[...] = o.reshape(P, _TCONV_NB, D) + b_ref[...][:, None, None]


def _k4b(t, w, b):
    return pl.pallas_call(
        _tconv_body,
        grid=(N // _TCONV_NB,),
        in_specs=[
            pl.BlockSpec((P, _TCONV_NB, D), lambda n: (0, n, 0)),
            pl.BlockSpec((P, P), lambda n: (0, 0)),
            pl.BlockSpec((P,), lambda n: (0,)),
        ],
        out_specs=pl.BlockSpec((P, _TCONV_NB, D), lambda n: (0, n, 0)),
        out_shape=jax.ShapeDtypeStruct((P, N, D), jnp.float32),
    )(t, w, b)


_K45_NB = 1000


def _k4b5_body(t_ref, tw_ref, tb_ref, h0_ref, h1_ref, attn_ref,
               w1_ref, b1_ref, w2_ref, b2_ref, out_ref):
    tb = t_ref[...].reshape(P, _K45_NB * D)
    h2 = lax.dot_general(tw_ref[...], tb, (((1,), (0,)), ((), ())),
                         preferred_element_type=jnp.float32)
    h2 = h2.reshape(P, _K45_NB, D) + tb_ref[...][:, None, None]
    a = jax.nn.softmax(attn_ref[...], axis=0)
    h = h0_ref[...] * a[0, 0] + h1_ref[...] * a[1, 0] + h2 * a[2, 0]
    hf = h.reshape(P * _K45_NB, D)
    hp = jax.nn.relu(_mm_t(hf, w1_ref[...]) + b1_ref[...])
    o = _mm_t(hp, w2_ref[...]) + b2_ref[...]
    out_ref[...] = o.reshape(P, _K45_NB, D)


def _k4b5(t, tw, tbias, h0, h1, attn, w1, b1, w2, b2):
    full = lambda *shape: pl.BlockSpec(shape, lambda n: (0,) * len(shape))
    blk = pl.BlockSpec((P, _K45_NB, D), lambda n: (0, n, 0))
    return pl.pallas_call(
        _k4b5_body,
        grid=(N // _K45_NB,),
        in_specs=[blk, full(P, P), full(P,), blk, blk, full(3, 1),
                  full(D, D), full(D,), full(D, D), full(D,)],
        out_specs=blk,
        out_shape=jax.ShapeDtypeStruct((P, N, D), jnp.float32),
    )(t, tw, tbias, h0, h1, attn, w1, b1, w2, b2)


def kernel(x, edge_index, pos, in_w1, in_b1, in_w2, in_b2, pos_w1, pos_b1,
           pos_w2, pos_b2, attn, g_gamma, g_beta, t_gamma, t_beta, tconv_w,
           tconv_b, out_w1, out_b1, out_w2, out_b2):
    row = edge_index[0]
    col = edge_index[1]

    degp = _sc_deg(col).reshape(NC, N_PAD, DEG_W)
    dinv = _kdinv(degp)

    h0, g0 = _k12(x.reshape(P, N, D), pos,
                  in_w1, in_b1, in_w2, in_b2, pos_w1, pos_b1, pos_w2, pos_b2,
                  g_gamma[0], g_beta[0], dinv)
    cv0 = _sc_conv(g0.reshape(P * N, D), row, col)
    t0 = _k4a(cv0.reshape(P, N_PAD, D), t_gamma[0], t_beta[0], dinv)
    h1 = _k4b(t0, tconv_w[0], tconv_b[0])

    g1 = _k2(h1, g_gamma[1], g_beta[1], dinv)
    cv1 = _sc_conv(g1.reshape(P * N, D), row, col)
    t1 = _k4a(cv1.reshape(P, N_PAD, D), t_gamma[1], t_beta[1], dinv)

    return _k4b5(t1, tconv_w[1], tconv_b[1], h0, h1, attn,
                 out_w1, out_b1, out_w2, out_b2)
